# baseline (speedup 1.0000x reference)
import functools

import jax
import jax.numpy as jnp
from jax import lax
from jax.experimental import pallas as pl
from jax.experimental.pallas import tpu as pltpu
from jax.experimental.pallas import tpu_sc as plsc

_BATCH = 8
_NUM_ROIS = 100
_RES = 28
_NUM_CLASSES = 91
_ROWS = _BATCH * _NUM_ROIS
_ROWS_PER_W = 32
_ACTIVE_W = _ROWS // _ROWS_PER_W
_NC = 2


@functools.partial(
    pl.kernel,
    mesh=plsc.VectorSubcoreMesh(core_axis_name="c", subcore_axis_name="s"),
    out_type=jax.ShapeDtypeStruct((_BATCH, _NUM_ROIS, _RES, _RES), jnp.float32),
    scratch_types=[
        pltpu.VMEM((_ROWS_PER_W,), jnp.int32),
        pltpu.SemaphoreType.DMA,
    ],
)
def _sc_gather(masks_hbm, cls_hbm, out_hbm, cls_v, sem):
    wid = lax.axis_index("s") * _NC + lax.axis_index("c")

    @pl.when(wid < _ACTIVE_W)
    def _():
        base = wid * _ROWS_PER_W
        pltpu.sync_copy(cls_hbm.at[pl.ds(base, _ROWS_PER_W)], cls_v)
        copies = []
        for j in range(_ROWS_PER_W):
            if j % 16 == 0:
                chunk = cls_v[pl.ds(j, 16)]
            c = chunk[j % 16]
            row = base + j
            b = row // _NUM_ROIS
            r = row % _NUM_ROIS
            copies.append(pltpu.make_async_copy(
                masks_hbm.at[b, r, c], out_hbm.at[b, r], sem))
        for cp in copies:
            cp.start()
        for cp in copies:
            cp.wait()


def kernel(mask_outputs, class_indices):
    cls = class_indices.reshape(_ROWS).astype(jnp.int32)
    return _sc_gather(mask_outputs, cls)


# speedup vs baseline: 6.5379x; 6.5379x over previous
import functools

import jax
import jax.numpy as jnp
from jax.experimental import pallas as pl
from jax.experimental.pallas import tpu as pltpu

_BATCH = 8
_NUM_ROIS = 100
_RES = 28
_NUM_CLASSES = 91
_YX = _RES * _RES
_NXB = 4


def _body(cls_ref, in_ref, out_ref):
    cls = cls_ref[...]
    acc = in_ref[0]
    for c in range(1, _NUM_CLASSES):
        acc = jnp.where((cls == c)[None], in_ref[c], acc)
    out_ref[...] = acc


def kernel(mask_outputs, class_indices):
    planes = jnp.transpose(mask_outputs, (2, 3, 4, 0, 1)).reshape(
        _NUM_CLASSES, _YX, _BATCH, _NUM_ROIS)
    cls = class_indices.astype(jnp.int32)
    out = pl.pallas_call(
        _body,
        grid=(_YX // _NXB,),
        in_specs=[
            pl.BlockSpec((_BATCH, _NUM_ROIS), lambda i: (0, 0)),
            pl.BlockSpec((_NUM_CLASSES, _NXB, _BATCH, _NUM_ROIS),
                         lambda i: (0, i, 0, 0)),
        ],
        out_specs=pl.BlockSpec((_NXB, _BATCH, _NUM_ROIS),
                               lambda i: (i, 0, 0)),
        out_shape=jax.ShapeDtypeStruct((_YX, _BATCH, _NUM_ROIS), jnp.float32),
        compiler_params=pltpu.CompilerParams(
            dimension_semantics=("arbitrary",)),
    )(cls, planes)
    return jnp.transpose(out.reshape(_RES, _RES, _BATCH, _NUM_ROIS),
                         (2, 3, 0, 1))


# speedup vs baseline: 7.2163x; 1.1037x over previous
import functools

import jax
import jax.numpy as jnp
from jax import lax
from jax.experimental import pallas as pl
from jax.experimental.pallas import tpu as pltpu
from jax.experimental.pallas import tpu_sc as plsc

_BATCH = 8
_NUM_ROIS = 100
_RES = 28
_NUM_CLASSES = 91
_ROWS = _BATCH * _NUM_ROIS
_YX = _RES * _RES
_NW = 32
_BIG = 25
_SMALL = 24
_L = 16


@functools.partial(
    pl.kernel,
    mesh=plsc.VectorSubcoreMesh(core_axis_name="c", subcore_axis_name="s"),
    out_type=jax.ShapeDtypeStruct((_YX, _BATCH, _NUM_ROIS), jnp.float32),
    scratch_types=[
        pltpu.VMEM((_ROWS,), jnp.int32),
        pltpu.VMEM((_NUM_CLASSES, _BATCH, _NUM_ROIS), jnp.float32),
        pltpu.VMEM((_BIG, _BATCH, _NUM_ROIS), jnp.float32),
        pltpu.SemaphoreType.DMA,
    ],
    compiler_params=pltpu.CompilerParams(needs_layout_passes=False),
)
def _sc_gather(planes_hbm, cls_hbm, out_hbm, cls_v, stage_v, out_v, sem):
    wid = lax.axis_index("s") * 2 + lax.axis_index("c")
    base = wid * _BIG - jnp.maximum(wid - _NW // 2, 0)
    count = jnp.where(wid < _NW // 2, _BIG, _SMALL)
    pltpu.sync_copy(cls_hbm, cls_v)
    iota = lax.iota(jnp.int32, _L)

    def body(j, _):
        @pl.when(j < count)
        def _():
            pltpu.async_copy(planes_hbm.at[:, base + j], stage_v, sem).wait()
            jvec = iota * 0 + j
            for k in range(_ROWS // _L):
                pos = iota + (k * _L)
                b0 = (k * _L) // _NUM_ROIS
                bvec = jnp.where(pos >= (b0 + 1) * _NUM_ROIS, b0 + 1, b0)
                rvec = pos - bvec * _NUM_ROIS
                vals = plsc.load_gather(
                    stage_v, [cls_v[pl.ds(k * _L, _L)], bvec, rvec])
                plsc.store_scatter(out_v, [jvec, bvec, rvec], vals)
        return 0

    lax.fori_loop(0, _BIG, body, 0)

    @pl.when(wid < _NW // 2)
    def _():
        pltpu.sync_copy(out_v, out_hbm.at[pl.ds(base, _BIG)])

    @pl.when(wid >= _NW // 2)
    def _():
        pltpu.sync_copy(out_v.at[pl.ds(0, _SMALL)],
                        out_hbm.at[pl.ds(base, _SMALL)])


def kernel(mask_outputs, class_indices):
    planes = jnp.transpose(mask_outputs, (2, 3, 4, 0, 1)).reshape(
        _NUM_CLASSES, _YX, _BATCH, _NUM_ROIS)
    cls = class_indices.reshape(_ROWS).astype(jnp.int32)
    out = _sc_gather(planes, cls)
    return jnp.transpose(out.reshape(_RES, _RES, _BATCH, _NUM_ROIS),
                         (2, 3, 0, 1))


# speedup vs baseline: 8.5320x; 1.1823x over previous
import functools

import jax
import jax.numpy as jnp
from jax import lax
from jax.experimental import pallas as pl
from jax.experimental.pallas import tpu as pltpu
from jax.experimental.pallas import tpu_sc as plsc

_BATCH = 8
_NUM_ROIS = 100
_RES = 28
_NUM_CLASSES = 91
_ROWS = _BATCH * _NUM_ROIS
_YX = _RES * _RES
_NW = 32
_SC_PER_W = 13
_SC_YX = _NW * _SC_PER_W
_TC_YX = _YX - _SC_YX
_NXB = 4
_L = 16


@functools.partial(
    pl.kernel,
    mesh=plsc.VectorSubcoreMesh(core_axis_name="c", subcore_axis_name="s"),
    out_type=jax.ShapeDtypeStruct((_SC_YX, _BATCH, _NUM_ROIS), jnp.float32),
    scratch_types=[
        pltpu.VMEM((_ROWS,), jnp.int32),
        pltpu.VMEM((_NUM_CLASSES, _BATCH, _NUM_ROIS), jnp.float32),
        pltpu.VMEM((_SC_PER_W, _BATCH, _NUM_ROIS), jnp.float32),
        pltpu.SemaphoreType.DMA,
    ],
    compiler_params=pltpu.CompilerParams(needs_layout_passes=False),
)
def _sc_gather(planes_hbm, cls_hbm, out_hbm, cls_v, stage_v, out_v, sem):
    wid = lax.axis_index("s") * 2 + lax.axis_index("c")
    base = wid * _SC_PER_W
    pltpu.sync_copy(cls_hbm, cls_v)
    iota = lax.iota(jnp.int32, _L)

    def body(j, _):
        pltpu.async_copy(planes_hbm.at[:, _TC_YX + base + j], stage_v,
                         sem).wait()
        jvec = iota * 0 + j
        for k in range(_ROWS // _L):
            pos = iota + (k * _L)
            b0 = (k * _L) // _NUM_ROIS
            bvec = jnp.where(pos >= (b0 + 1) * _NUM_ROIS, b0 + 1, b0)
            rvec = pos - bvec * _NUM_ROIS
            vals = plsc.load_gather(
                stage_v, [cls_v[pl.ds(k * _L, _L)], bvec, rvec])
            plsc.store_scatter(out_v, [jvec, bvec, rvec], vals)
        return 0

    lax.fori_loop(0, _SC_PER_W, body, 0)
    pltpu.sync_copy(out_v, out_hbm.at[pl.ds(base, _SC_PER_W)])


def _tc_body(cls_ref, in_ref, out_ref):
    cls = cls_ref[...]
    acc = in_ref[0]
    for c in range(1, _NUM_CLASSES):
        acc = jnp.where((cls == c)[None], in_ref[c], acc)
    out_ref[...] = acc


def kernel(mask_outputs, class_indices):
    planes = jnp.transpose(mask_outputs, (2, 3, 4, 0, 1)).reshape(
        _NUM_CLASSES, _YX, _BATCH, _NUM_ROIS)
    cls2 = class_indices.astype(jnp.int32)
    cls1 = class_indices.reshape(_ROWS).astype(jnp.int32)
    out_sc = _sc_gather(planes, cls1)
    out_tc = pl.pallas_call(
        _tc_body,
        grid=(_TC_YX // _NXB,),
        in_specs=[
            pl.BlockSpec((_BATCH, _NUM_ROIS), lambda i: (0, 0)),
            pl.BlockSpec((_NUM_CLASSES, _NXB, _BATCH, _NUM_ROIS),
                         lambda i: (0, i, 0, 0)),
        ],
        out_specs=pl.BlockSpec((_NXB, _BATCH, _NUM_ROIS),
                               lambda i: (i, 0, 0)),
        out_shape=jax.ShapeDtypeStruct((_TC_YX, _BATCH, _NUM_ROIS),
                                       jnp.float32),
        compiler_params=pltpu.CompilerParams(
            dimension_semantics=("arbitrary",)),
    )(cls2, planes)
    out = jnp.concatenate([out_tc, out_sc], axis=0)
    return jnp.transpose(out.reshape(_RES, _RES, _BATCH, _NUM_ROIS),
                         (2, 3, 0, 1))


# speedup vs baseline: 8.9018x; 1.0433x over previous
import functools

import jax
import jax.numpy as jnp
from jax import lax
from jax.experimental import pallas as pl
from jax.experimental.pallas import tpu as pltpu
from jax.experimental.pallas import tpu_sc as plsc

_BATCH = 8
_NUM_ROIS = 100
_RES = 28
_NUM_CLASSES = 91
_ROWS = _BATCH * _NUM_ROIS
_YX = _RES * _RES
_NW = 32
_SC_PER_W = 14
_SC_YX = _NW * _SC_PER_W
_TC_YX = _YX - _SC_YX
_NXB = 4
_L = 16


@functools.partial(
    pl.kernel,
    mesh=plsc.VectorSubcoreMesh(core_axis_name="c", subcore_axis_name="s"),
    out_type=jax.ShapeDtypeStruct((_SC_YX, _BATCH, _NUM_ROIS), jnp.float32),
    scratch_types=[
        pltpu.VMEM((_ROWS,), jnp.int32),
        pltpu.VMEM((_NUM_CLASSES, _BATCH, _NUM_ROIS), jnp.float32),
        pltpu.VMEM((_SC_PER_W, _BATCH, _NUM_ROIS), jnp.float32),
        pltpu.SemaphoreType.DMA,
    ],
    compiler_params=pltpu.CompilerParams(needs_layout_passes=False),
)
def _sc_gather(planes_hbm, cls_hbm, out_hbm, cls_v, stage_v, out_v, sem):
    wid = lax.axis_index("s") * 2 + lax.axis_index("c")
    base = wid * _SC_PER_W
    pltpu.sync_copy(cls_hbm, cls_v)
    iota = lax.iota(jnp.int32, _L)

    def body(j, _):
        pltpu.async_copy(planes_hbm.at[:, _TC_YX + base + j], stage_v,
                         sem).wait()
        jvec = iota * 0 + j
        for k in range(_ROWS // _L):
            pos = iota + (k * _L)
            b0 = (k * _L) // _NUM_ROIS
            bvec = jnp.where(pos >= (b0 + 1) * _NUM_ROIS, b0 + 1, b0)
            rvec = pos - bvec * _NUM_ROIS
            vals = plsc.load_gather(
                stage_v, [cls_v[pl.ds(k * _L, _L)], bvec, rvec])
            plsc.store_scatter(out_v, [jvec, bvec, rvec], vals)
        return 0

    lax.fori_loop(0, _SC_PER_W, body, 0)
    pltpu.sync_copy(out_v, out_hbm.at[pl.ds(base, _SC_PER_W)])


def _tc_body(cls_ref, in_ref, out_ref):
    cls = cls_ref[...]
    acc = in_ref[0]
    for c in range(1, _NUM_CLASSES):
        acc = jnp.where((cls == c)[None], in_ref[c], acc)
    out_ref[...] = acc


def kernel(mask_outputs, class_indices):
    planes = jnp.transpose(mask_outputs, (2, 3, 4, 0, 1)).reshape(
        _NUM_CLASSES, _YX, _BATCH, _NUM_ROIS)
    cls2 = class_indices.astype(jnp.int32)
    cls1 = class_indices.reshape(_ROWS).astype(jnp.int32)
    out_sc = _sc_gather(planes, cls1)
    out_tc = pl.pallas_call(
        _tc_body,
        grid=(_TC_YX // _NXB,),
        in_specs=[
            pl.BlockSpec((_BATCH, _NUM_ROIS), lambda i: (0, 0)),
            pl.BlockSpec((_NUM_CLASSES, _NXB, _BATCH, _NUM_ROIS),
                         lambda i: (0, i, 0, 0)),
        ],
        out_specs=pl.BlockSpec((_NXB, _BATCH, _NUM_ROIS),
                               lambda i: (i, 0, 0)),
        out_shape=jax.ShapeDtypeStruct((_TC_YX, _BATCH, _NUM_ROIS),
                                       jnp.float32),
        compiler_params=pltpu.CompilerParams(
            dimension_semantics=("arbitrary",)),
    )(cls2, planes)
    out = jnp.concatenate([out_tc, out_sc], axis=0)
    return jnp.transpose(out.reshape(_RES, _RES, _BATCH, _NUM_ROIS),
                         (2, 3, 0, 1))
